# baseline (speedup 1.0000x reference)
import jax
import jax.numpy as jnp
from jax import lax
from jax.experimental import pallas as pl
from jax.experimental.pallas import tpu as pltpu
from jax.experimental.pallas import tpu_sc as plsc

B = 16384
L = 50
DIM = 32
VOCAB = 1000000
CHUNK = 128
NB = B // CHUNK
NCH = L * NB
NW = 32
CPW = NCH // NW
NBUF = 8
QD = NBUF // 2


def _body(idx_hbm, tok_hbm, pos_hbm, out_hbm,
          idx_v, pos_v,
          a0, a1, a2, a3, a4, a5, a6, a7,
          g0, g1, g2, g3, g4, g5, g6, g7,
          s0, s1, s2, s3, s4, s5, s6, s7):
    abuf = (a0, a1, a2, a3, a4, a5, a6, a7)
    gsem = (g0, g1, g2, g3, g4, g5, g6, g7)
    ssem = (s0, s1, s2, s3, s4, s5, s6, s7)

    wid = lax.axis_index("s") * 2 + lax.axis_index("c")
    base = wid * CPW
    l0 = lax.shift_right_logical(base, 7)

    pltpu.sync_copy(idx_hbm.at[pl.ds(base, CPW)], idx_v)
    pltpu.sync_copy(pos_hbm.at[pl.ds(l0, 4)], pos_v)

    def start_gather(j, b):
        pltpu.make_async_copy(tok_hbm.at[idx_v.at[j]], abuf[b], gsem[b]).start()

    def wait_gather(j, b):
        pltpu.make_async_copy(tok_hbm.at[idx_v.at[j]], abuf[b], gsem[b]).wait()

    def chunk_lbb(j):
        c = base + j
        return lax.shift_right_logical(c, 7), lax.bitwise_and(c, 127)

    def start_store(j, b):
        l, bb = chunk_lbb(j)
        pltpu.make_async_copy(
            abuf[b],
            out_hbm.at[l, pl.ds(bb * CHUNK, CHUNK), pl.ds(0, DIM)],
            ssem[b]).start()

    def wait_store(j, b):
        l, bb = chunk_lbb(j)
        pltpu.make_async_copy(
            abuf[b],
            out_hbm.at[l, pl.ds(bb * CHUNK, CHUNK), pl.ds(0, DIM)],
            ssem[b]).wait()

    for bi in range(QD):
        start_gather(bi, bi)

    def add_pos(j, b):
        l, _ = chunk_lbb(j)
        lblk = l - l0
        a = abuf[b]
        p0 = pos_v[lblk, pl.ds(0, 16)]
        p1 = pos_v[lblk, pl.ds(16, 16)]

        def rowfn(t, carry):
            a[t, pl.ds(0, 16)] = a[t, pl.ds(0, 16)] + p0
            a[t, pl.ds(16, 16)] = a[t, pl.ds(16, 16)] + p1
            return carry
        lax.fori_loop(0, CHUNK, rowfn, None)

    def step(t, carry):
        for bi in range(NBUF):
            j = t * NBUF + bi
            wait_gather(j, bi)
            add_pos(j, bi)
            start_store(j, bi)
            jn = j + QD
            bn = (bi + QD) % NBUF

            @pl.when(jnp.logical_and(jn < CPW, j >= QD))
            def _drain():
                wait_store(j - QD, bn)

            @pl.when(jn < CPW)
            def _refill():
                start_gather(jn, bn)
        return carry

    lax.fori_loop(0, CPW // NBUF, step, None)

    for bi in range(NBUF):
        wait_store(CPW - NBUF + bi, bi)


def kernel(input_ids, token_table, pos_table):
    idx4 = input_ids.astype(jnp.int32).T.reshape(NCH, CHUNK) * 4
    tokp = jnp.pad(token_table, ((0, 0), (0, 96))).reshape(4 * VOCAB, DIM)
    posp = jnp.pad(pos_table, ((0, 6), (0, 0)))
    mesh = plsc.VectorSubcoreMesh(core_axis_name="c", subcore_axis_name="s")
    out = pl.kernel(
        _body,
        out_type=jax.ShapeDtypeStruct((L, B, 128), jnp.float32),
        mesh=mesh,
        compiler_params=pltpu.CompilerParams(use_tc_tiling_on_sc=False),
        scratch_types=(
            [pltpu.VMEM((CPW, CHUNK), jnp.int32),
             pltpu.VMEM((4, DIM), jnp.float32)]
            + [pltpu.VMEM((CHUNK, DIM), jnp.float32) for _ in range(NBUF)]
            + [pltpu.SemaphoreType.DMA for _ in range(2 * NBUF)]
        ),
    )(idx4, tokp, posp)
    return out[:, :, :DIM].transpose(1, 0, 2)


# speedup vs baseline: 1.4471x; 1.4471x over previous
import jax
import jax.numpy as jnp
from jax import lax
from jax.experimental import pallas as pl
from jax.experimental.pallas import tpu as pltpu
from jax.experimental.pallas import tpu_sc as plsc

B = 16384
L = 50
DIM = 32
VOCAB = 1000000
CHUNK = 128
NB = B // CHUNK
NCH = L * NB
NW = 32
CPW = NCH // NW
NBUF = 8
QD = NBUF // 2


def _body(idx_hbm, tok_hbm, pos_hbm, out_hbm,
          idx_v, pos_v,
          a0, a1, a2, a3, a4, a5, a6, a7,
          g0, g1, g2, g3, g4, g5, g6, g7,
          s0, s1, s2, s3, s4, s5, s6, s7):
    abuf = (a0, a1, a2, a3, a4, a5, a6, a7)
    gsem = (g0, g1, g2, g3, g4, g5, g6, g7)
    ssem = (s0, s1, s2, s3, s4, s5, s6, s7)

    wid = lax.axis_index("s") * 2 + lax.axis_index("c")
    base = wid * CPW
    l0 = lax.shift_right_logical(base, 7)

    pltpu.sync_copy(idx_hbm.at[pl.ds(base, CPW)], idx_v)
    pltpu.sync_copy(pos_hbm.at[pl.ds(l0, 4)], pos_v)

    def start_gather(j, b):
        pltpu.make_async_copy(tok_hbm.at[idx_v.at[j]], abuf[b], gsem[b]).start()

    def wait_gather(j, b):
        pltpu.make_async_copy(tok_hbm.at[idx_v.at[j]], abuf[b], gsem[b]).wait()

    def chunk_lbb(j):
        c = base + j
        return lax.shift_right_logical(c, 7), lax.bitwise_and(c, 127)

    def start_store(j, b):
        l, bb = chunk_lbb(j)
        for g in range(16):
            pltpu.make_async_copy(
                abuf[b].at[pl.ds(8 * g, 8)],
                out_hbm.at[l, bb * 16 + g, :, pl.ds(0, DIM)],
                ssem[b]).start()

    def wait_store(j, b):
        l, bb = chunk_lbb(j)
        for g in range(16):
            pltpu.make_async_copy(
                abuf[b].at[pl.ds(8 * g, 8)],
                out_hbm.at[l, bb * 16 + g, :, pl.ds(0, DIM)],
                ssem[b]).wait()

    for bi in range(QD):
        start_gather(bi, bi)

    def add_pos(j, b):
        l, _ = chunk_lbb(j)
        lblk = l - l0
        a = abuf[b]
        p0 = pos_v[lblk, pl.ds(0, 16)]
        p1 = pos_v[lblk, pl.ds(16, 16)]

        def rowfn(r, carry):
            t0 = r * 8
            for k in range(8):
                a[t0 + k, pl.ds(0, 16)] = a[t0 + k, pl.ds(0, 16)] + p0
                a[t0 + k, pl.ds(16, 16)] = a[t0 + k, pl.ds(16, 16)] + p1
            return carry
        lax.fori_loop(0, CHUNK // 8, rowfn, None)

    def step(t, carry):
        for bi in range(NBUF):
            j = t * NBUF + bi
            wait_gather(j, bi)
            add_pos(j, bi)
            start_store(j, bi)
            jn = j + QD
            bn = (bi + QD) % NBUF

            @pl.when(jnp.logical_and(jn < CPW, j >= QD))
            def _drain():
                wait_store(j - QD, bn)

            @pl.when(jn < CPW)
            def _refill():
                start_gather(jn, bn)
        return carry

    lax.fori_loop(0, CPW // NBUF, step, None)

    for bi in range(NBUF):
        wait_store(CPW - NBUF + bi, bi)


def kernel(input_ids, token_table, pos_table):
    idx4 = input_ids.astype(jnp.int32).T.reshape(NCH, CHUNK) * 4
    tokp = jnp.pad(token_table, ((0, 0), (0, 96))).reshape(4 * VOCAB, DIM)
    posp = jnp.pad(pos_table, ((0, 6), (0, 0)))
    mesh = plsc.VectorSubcoreMesh(core_axis_name="c", subcore_axis_name="s")
    out = pl.kernel(
        _body,
        out_type=jax.ShapeDtypeStruct((L, B // 8, 8, 128), jnp.float32),
        mesh=mesh,
        compiler_params=pltpu.CompilerParams(use_tc_tiling_on_sc=False),
        scratch_types=(
            [pltpu.VMEM((CPW, CHUNK), jnp.int32),
             pltpu.VMEM((4, DIM), jnp.float32)]
            + [pltpu.VMEM((CHUNK, DIM), jnp.float32) for _ in range(NBUF)]
            + [pltpu.SemaphoreType.DMA for _ in range(2 * NBUF)]
        ),
    )(idx4, tokp, posp)
    return out.reshape(L, B, 128)[:, :, :DIM].transpose(1, 0, 2)
